# Initial kernel scaffold; baseline (speedup 1.0000x reference)
import jax
import jax.numpy as jnp
from jax.experimental import pallas as pl


def kernel(x, e, edge_index, pe, params):
    raise NotImplementedError("write your pallas kernel here")


# speedup vs baseline: 2.1213x; 2.1213x over previous
import functools

import jax
import jax.numpy as jnp
from jax import lax
from jax.experimental import pallas as pl
from jax.experimental.pallas import tpu as pltpu
from jax.experimental.pallas import tpu_sc as plsc

N = 10000
NH = N // 2
E = 320000
D = 128
NL = 4

BN = 1000
BE = 2000

NSUB = 16
C = 80
PC = NSUB * C
NB = 6
NPH = NB // 2
E_PAD = E + NB * PC
DUMMY_DST = 2 * N
NQ = 1672
N_LAST = N - (NB - 1) * NQ
ACC_R = NQ + 8
RZ = 104
ZB = 120

_f32 = jnp.float32


def _nm_body(xr, per, w4r, b4r, w2r, b2r, axr, bxr, dxr, exr, aper, bpr):
    xpe = jnp.concatenate([xr[...], per[...]], axis=1)
    y = jnp.dot(xpe, w4r[...], preferred_element_type=_f32) + b4r[...]
    p = jnp.dot(per[...], w2r[...], preferred_element_type=_f32) + b2r[...]
    axr[...] = y[:, 0:D]
    bxr[...] = y[:, D:2 * D]
    dxr[...] = y[:, 2 * D:3 * D]
    exr[...] = y[:, 3 * D:4 * D]
    aper[...] = p[:, 0:D]
    bpr[...] = p[:, D:2 * D]


def _node_matmul(x, pe, w4, b4, w2, b2):
    b2d = lambda: pl.BlockSpec((BN, D), lambda i: (i, 0))
    outs = [jax.ShapeDtypeStruct((N, D), _f32) for _ in range(6)]
    return pl.pallas_call(
        _nm_body, grid=(N // BN,),
        in_specs=[b2d(), b2d(),
                  pl.BlockSpec((2 * D, 4 * D), lambda i: (0, 0)),
                  pl.BlockSpec((1, 4 * D), lambda i: (0, 0)),
                  pl.BlockSpec((D, 2 * D), lambda i: (0, 0)),
                  pl.BlockSpec((1, 2 * D), lambda i: (0, 0))],
        out_specs=[b2d() for _ in range(6)],
        out_shape=outs,
    )(x, pe, w4, b4, w2, b2)


def _ce0_body(er, wcr, bcr, cer):
    cer[...] = jnp.dot(er[...], wcr[...], preferred_element_type=_f32) + bcr[...]


def _ce0(e, wc, bc):
    return pl.pallas_call(
        _ce0_body, grid=(E // BE,),
        in_specs=[pl.BlockSpec((BE, D), lambda i: (i, 0)),
                  pl.BlockSpec((D, D), lambda i: (0, 0)),
                  pl.BlockSpec((1, D), lambda i: (0, 0))],
        out_specs=[pl.BlockSpec((BE, D), lambda i: (i, 0))],
        out_shape=[jax.ShapeDtypeStruct((E, D), _f32)],
    )(e, wc, bc)[0]


def _ebn_apply(eijr, str_, gr, br, einr):
    mu = str_[0:1, :] * (1.0 / E)
    var = str_[1:2, :] * (1.0 / E) - mu * mu
    xn = (eijr[...] - mu) * lax.rsqrt(var + 1e-5) * gr[...] + br[...]
    return einr[...] + jnp.maximum(xn, 0.0)


def _est_body(eijr, str_):
    i = pl.program_id(0)

    @pl.when(i == 0)
    def _init():
        str_[...] = jnp.zeros_like(str_)

    eij = eijr[...]
    str_[0:1, :] += jnp.sum(eij, axis=0, keepdims=True)
    str_[1:2, :] += jnp.sum(eij * eij, axis=0, keepdims=True)


def _estats(eij):
    return pl.pallas_call(
        _est_body, grid=(E // BE,),
        in_specs=[pl.BlockSpec((BE, D), lambda i: (i, 0))],
        out_specs=[pl.BlockSpec((2, D), lambda i: (0, 0))],
        out_shape=[jax.ShapeDtypeStruct((2, D), _f32)],
    )(eij)[0]


def _efuse_body(eijr, str_, einr, gr, br, wcr, bcr, eor, cer):
    e_out = _ebn_apply(eijr[...], str_, gr, br, einr)
    eor[...] = e_out
    cer[...] = jnp.dot(e_out, wcr[...], preferred_element_type=_f32) + bcr[...]


def _efuse(eij, st3, e_in, g, b, wc, bc):
    return pl.pallas_call(
        _efuse_body, grid=(E // BE,),
        in_specs=[pl.BlockSpec((BE, D), lambda i: (i, 0)),
                  pl.BlockSpec((2, D), lambda i: (0, 0)),
                  pl.BlockSpec((BE, D), lambda i: (i, 0)),
                  pl.BlockSpec((1, D), lambda i: (0, 0)),
                  pl.BlockSpec((1, D), lambda i: (0, 0)),
                  pl.BlockSpec((D, D), lambda i: (0, 0)),
                  pl.BlockSpec((1, D), lambda i: (0, 0))],
        out_specs=[pl.BlockSpec((BE, D), lambda i: (i, 0)),
                   pl.BlockSpec((BE, D), lambda i: (i, 0))],
        out_shape=[jax.ShapeDtypeStruct((E, D), _f32),
                   jax.ShapeDtypeStruct((E, D), _f32)],
    )(eij, st3, e_in, g, b, wc, bc)


def _efinal_body(eijr, str_, einr, gr, br, eor):
    eor[...] = _ebn_apply(eijr[...], str_, gr, br, einr)


def _efinal(eij, st3, e_in, g, b):
    return pl.pallas_call(
        _efinal_body, grid=(E // BE,),
        in_specs=[pl.BlockSpec((BE, D), lambda i: (i, 0)),
                  pl.BlockSpec((2, D), lambda i: (0, 0)),
                  pl.BlockSpec((BE, D), lambda i: (i, 0)),
                  pl.BlockSpec((1, D), lambda i: (0, 0)),
                  pl.BlockSpec((1, D), lambda i: (0, 0))],
        out_specs=[pl.BlockSpec((BE, D), lambda i: (i, 0))],
        out_shape=[jax.ShapeDtypeStruct((E, D), _f32)],
    )(eij, st3, e_in, g, b)[0]


def _xst_body(axr, numr, denr, str_):
    i = pl.program_id(0)

    @pl.when(i == 0)
    def _init():
        str_[...] = jnp.zeros_like(str_)

    xpre = axr[...] + numr[...] / (denr[...] + 1e-6)
    str_[0:1, :] += jnp.sum(xpre, axis=0, keepdims=True)
    str_[1:2, :] += jnp.sum(xpre * xpre, axis=0, keepdims=True)


def _xstats(ax, num, den):
    b2d = lambda: pl.BlockSpec((BN, D), lambda i: (i, 0))
    return pl.pallas_call(
        _xst_body, grid=(N // BN,),
        in_specs=[b2d(), b2d(), b2d()],
        out_specs=[pl.BlockSpec((2, D), lambda i: (0, 0))],
        out_shape=[jax.ShapeDtypeStruct((2, D), _f32)],
    )(ax, num, den)[0]


def _xap_body(axr, aper, numr, denr, numpr, str_, xinr, peinr, gr, br, xor_, peor):
    den = denr[...] + 1e-6
    xpre = axr[...] + numr[...] / den
    mu = str_[0:1, :] * (1.0 / N)
    var = str_[1:2, :] * (1.0 / N) - mu * mu
    xn = (xpre - mu) * lax.rsqrt(var + 1e-5) * gr[...] + br[...]
    xor_[...] = xinr[...] + jnp.maximum(xn, 0.0)
    peor[...] = peinr[...] + jnp.tanh(aper[...] + numpr[...] / den)


def _xapply(ax, ape, num, den, nump, xst, x_in, pe_in, g, b):
    b2d = lambda: pl.BlockSpec((BN, D), lambda i: (i, 0))
    return pl.pallas_call(
        _xap_body, grid=(N // BN,),
        in_specs=[b2d(), b2d(), b2d(), b2d(), b2d(),
                  pl.BlockSpec((2, D), lambda i: (0, 0)),
                  b2d(), b2d(),
                  pl.BlockSpec((1, D), lambda i: (0, 0)),
                  pl.BlockSpec((1, D), lambda i: (0, 0))],
        out_specs=[b2d(), b2d()],
        out_shape=[jax.ShapeDtypeStruct((N, D), _f32),
                   jax.ShapeDtypeStruct((N, D), _f32)],
    )(ax, ape, num, den, nump, xst, x_in, pe_in, g, b)


def _sc_edge(dst_p, src_p, eid_p, offs, dx, ex, bx, bp, ce):
    mesh = plsc.VectorSubcoreMesh(core_axis_name="c", subcore_axis_name="s")

    @functools.partial(
        pl.kernel, mesh=mesh,
        out_type=[jax.ShapeDtypeStruct((E + 8, D), _f32),
                  jax.ShapeDtypeStruct((N, D), _f32),
                  jax.ShapeDtypeStruct((N, D), _f32),
                  jax.ShapeDtypeStruct((N, D), _f32)],
        scratch_types=[pltpu.VMEM((16,), jnp.int32),
                       pltpu.VMEM((C,), jnp.int32),
                       pltpu.VMEM((C,), jnp.int32),
                       pltpu.VMEM((C,), jnp.int32),
                       pltpu.VMEM((C,), jnp.int32),
                       pltpu.VMEM((C,), jnp.int32),
                       pltpu.VMEM((C,), jnp.int32),
                       pltpu.VMEM((C, D), _f32),
                       pltpu.VMEM((C, D), _f32),
                       pltpu.VMEM((C, D), _f32),
                       pltpu.VMEM((C, D), _f32),
                       pltpu.VMEM((C, D), _f32),
                       pltpu.VMEM((ZB, D), _f32),
                       pltpu.VMEM_SHARED((ACC_R, D), _f32),
                       pltpu.VMEM_SHARED((ACC_R, D), _f32),
                       pltpu.VMEM_SHARED((ACC_R, D), _f32),
                       pltpu.SemaphoreType.DMA,
                       pltpu.SemaphoreType.DMA,
                       pltpu.SemaphoreType.DMA,
                       pltpu.SemaphoreType.DMA,
                       pltpu.SemaphoreType.DMA])
    def k(dst_h, src_h, eid_h, offs_h, dx_h, ex_h, bx_h, bp_h, ce_h,
          eij_h, num_h, den_h, nump_h,
          offv, dstb, dstlb, dstgb, srcb, eidb, eidgb,
          dxb, exb, bxb, bpb, ceb, zb,
          accn, accd, accp, sm0, sm1, sm2, sm3, sm4):
        c = lax.axis_index("c")
        s = lax.axis_index("s")

        def zrow(r, _):
            for k8 in range(D // 16):
                zb[r, pl.ds(16 * k8, 16)] = jnp.zeros((16,), _f32)
            return 0

        lax.fori_loop(0, ZB, zrow, 0)
        pltpu.sync_copy(offs_h, offv)
        ov = offv[...]

        for p in range(NPH):
            start_b = jnp.where(c == 0, ov[2 * p], ov[2 * (NPH + p)])
            len_b = jnp.where(c == 0, ov[2 * p + 1], ov[2 * (NPH + p) + 1])
            qbase = (NPH * c + p) * NQ
            tlen = len_b // NSUB
            nch = tlen // C
            tstart = start_b + s * tlen

            pltpu.sync_copy(zb.at[pl.ds(0, RZ)], accn.at[pl.ds(s * RZ, RZ)])
            pltpu.sync_copy(zb.at[pl.ds(0, RZ)], accd.at[pl.ds(s * RZ, RZ)])
            pltpu.sync_copy(zb.at[pl.ds(0, RZ)], accp.at[pl.ds(s * RZ, RZ)])

            @pl.when(s == NSUB - 1)
            def _ztail():
                tb = NSUB * RZ
                pltpu.sync_copy(zb.at[pl.ds(0, ACC_R - NSUB * RZ)],
                                accn.at[pl.ds(tb, ACC_R - NSUB * RZ)])
                pltpu.sync_copy(zb.at[pl.ds(0, ACC_R - NSUB * RZ)],
                                accd.at[pl.ds(tb, ACC_R - NSUB * RZ)])
                pltpu.sync_copy(zb.at[pl.ds(0, ACC_R - NSUB * RZ)],
                                accp.at[pl.ds(tb, ACC_R - NSUB * RZ)])

            plsc.subcore_barrier()

            def chunk(j, _):
                base = pl.multiple_of(tstart + j * C, 16)
                pltpu.sync_copy(dst_h.at[pl.ds(base, C)], dstb)
                pltpu.sync_copy(src_h.at[pl.ds(base, C)], srcb)
                pltpu.sync_copy(eid_h.at[pl.ds(base, C)], eidb)

                def adj(r, _2):
                    sl = pl.ds(16 * r, 16)
                    dv = dstb[sl]
                    dstlb[sl] = jnp.minimum(dv - qbase, NQ)
                    dstgb[sl] = jnp.minimum(dv, N - 1)
                    eidgb[sl] = jnp.minimum(eidb[sl], E - 1)
                    return 0

                lax.fori_loop(0, C // 16, adj, 0)
                h0 = pltpu.async_copy(dx_h.at[dstgb], dxb, sm0)
                h1 = pltpu.async_copy(ex_h.at[srcb], exb, sm1)
                h2 = pltpu.async_copy(bx_h.at[srcb], bxb, sm2)
                h3 = pltpu.async_copy(bp_h.at[srcb], bpb, sm3)
                h4 = pltpu.async_copy(ce_h.at[eidgb], ceb, sm4)
                h0.wait()
                h1.wait()
                h2.wait()
                h3.wait()
                h4.wait()

                def row(r, _2):
                    for k8 in range(D // 16):
                        sl = pl.ds(16 * k8, 16)
                        eij = dxb[r, sl] + exb[r, sl] + ceb[r, sl]
                        ceb[r, sl] = eij
                        sg = 1.0 / (1.0 + jnp.exp(-eij))
                        dxb[r, sl] = sg
                        bxb[r, sl] = sg * bxb[r, sl]
                        bpb[r, sl] = sg * bpb[r, sl]
                    return 0

                lax.fori_loop(0, C, row, 0)
                h5 = pltpu.async_copy(ceb, eij_h.at[eidb], sm4)
                h5.wait()
                pltpu.sync_copy(bxb, accn.at[dstlb], add=True)
                pltpu.sync_copy(dxb, accd.at[dstlb], add=True)
                pltpu.sync_copy(bpb, accp.at[dstlb], add=True)
                return 0

            lax.fori_loop(0, nch, chunk, 0)
            plsc.subcore_barrier()

            @pl.when(s < NSUB - 1)
            def _cmain():
                pltpu.sync_copy(accn.at[pl.ds(s * RZ, RZ)],
                                num_h.at[pl.ds(qbase + s * RZ, RZ)])
                pltpu.sync_copy(accd.at[pl.ds(s * RZ, RZ)],
                                den_h.at[pl.ds(qbase + s * RZ, RZ)])
                pltpu.sync_copy(accp.at[pl.ds(s * RZ, RZ)],
                                nump_h.at[pl.ds(qbase + s * RZ, RZ)])

            tb = (NSUB - 1) * RZ
            nfull = NQ - tb
            nlast = N_LAST - tb
            if p < NPH - 1:
                @pl.when(s == NSUB - 1)
                def _ctail():
                    pltpu.sync_copy(accn.at[pl.ds(tb, nfull)],
                                    num_h.at[pl.ds(qbase + tb, nfull)])
                    pltpu.sync_copy(accd.at[pl.ds(tb, nfull)],
                                    den_h.at[pl.ds(qbase + tb, nfull)])
                    pltpu.sync_copy(accp.at[pl.ds(tb, nfull)],
                                    nump_h.at[pl.ds(qbase + tb, nfull)])
            else:
                @pl.when(jnp.logical_and(s == NSUB - 1, c == 0))
                def _ctailf():
                    pltpu.sync_copy(accn.at[pl.ds(tb, nfull)],
                                    num_h.at[pl.ds(qbase + tb, nfull)])
                    pltpu.sync_copy(accd.at[pl.ds(tb, nfull)],
                                    den_h.at[pl.ds(qbase + tb, nfull)])
                    pltpu.sync_copy(accp.at[pl.ds(tb, nfull)],
                                    nump_h.at[pl.ds(qbase + tb, nfull)])

                @pl.when(jnp.logical_and(s == NSUB - 1, c == 1))
                def _ctaill():
                    pltpu.sync_copy(accn.at[pl.ds(tb, nlast)],
                                    num_h.at[pl.ds(qbase + tb, nlast)])
                    pltpu.sync_copy(accd.at[pl.ds(tb, nlast)],
                                    den_h.at[pl.ds(qbase + tb, nlast)])
                    pltpu.sync_copy(accp.at[pl.ds(tb, nlast)],
                                    nump_h.at[pl.ds(qbase + tb, nlast)])

            if p < NPH - 1:
                plsc.subcore_barrier()

    return k(dst_p, src_p, eid_p, offs, dx, ex, bx, bp, ce)


def _partition_edges(src0, dst0):
    i32 = jnp.int32
    b = jnp.minimum(dst0 // NQ, NB - 1)
    pos = jnp.zeros((E,), i32)
    starts = []
    start = jnp.zeros((), i32)
    for k in range(NB):
        ink = (b == k).astype(i32)
        rank = jnp.cumsum(ink) - ink
        pos = jnp.where(ink == 1, start + rank, pos)
        cnt = jnp.sum(ink)
        plen = ((cnt + PC - 1) // PC) * PC
        starts.append(start)
        starts.append(plen.astype(i32))
        start = start + plen
    src_p = jnp.zeros((E_PAD,), i32).at[pos].set(src0)
    dst_p = jnp.full((E_PAD,), DUMMY_DST, i32).at[pos].set(dst0)
    eid_p = jnp.full((E_PAD,), E, i32).at[pos].set(jnp.arange(E, dtype=i32))
    z = jnp.zeros((), i32)
    offs = jnp.stack(starts + [z] * (16 - len(starts)))
    return src_p, dst_p, eid_p, offs


def kernel(x, e, edge_index, pe, params):
    src0 = edge_index[0]
    dst0 = edge_index[1]
    src_p, dst_p, eid_p, offs = _partition_edges(src0, dst0)
    x_cur, pe_cur, e_cur = x, pe, e
    ceT = None
    for l in range(NL):
        w4 = jnp.concatenate([params['WA'][l], params['WB'][l],
                              params['WD'][l], params['WE'][l]], axis=1)
        b4 = jnp.concatenate([params['bA'][l], params['bB'][l],
                              params['bD'][l], params['bE'][l]], axis=0)[None, :]
        w2 = jnp.concatenate([params['WAp'][l], params['WBp'][l]], axis=1)
        b2 = jnp.concatenate([params['bAp'][l], params['bBp'][l]], axis=0)[None, :]
        if l == 0:
            ceT = _ce0(e_cur, params['WC'][0], params['bC'][0][None, :])
        ax, bx, dx, ex, ape, bp = _node_matmul(x_cur, pe_cur, w4, b4, w2, b2)
        eijF, num, den, nump = _sc_edge(dst_p, src_p, eid_p, offs,
                                        dx, ex, bx, bp, ceT)
        xst = _xstats(ax, num, den)
        x_new, pe_new = _xapply(ax, ape, num, den, nump, xst, x_cur, pe_cur,
                                params['bn_x_g'][l][None, :], params['bn_x_b'][l][None, :])
        st3 = _estats(eijF)
        eij = eijF
        g = params['bn_e_g'][l][None, :]
        b = params['bn_e_b'][l][None, :]
        if l < NL - 1:
            e_new, ceT = _efuse(eij, st3, e_cur,
                                g, b, params['WC'][l + 1], params['bC'][l + 1][None, :])
        else:
            e_new = _efinal(eij, st3, e_cur, g, b)
        x_cur, pe_cur, e_cur = x_new, pe_new, e_new
    return (x_cur, e_cur)


# speedup vs baseline: 2.3648x; 1.1148x over previous
import functools

import jax
import jax.numpy as jnp
from jax import lax
from jax.experimental import pallas as pl
from jax.experimental.pallas import tpu as pltpu
from jax.experimental.pallas import tpu_sc as plsc

N = 10000
E = 320000
D = 128
NL = 4

BN = 1000
BE = 2000

NSUB = 16
C = 64
PC = NSUB * C
PCP = 2 * PC
NB = 8
NPH = NB // 2
E_PAD = E + NB * PCP
DUMMY_DST = 2 * N
NQ = 1256
N_LAST = N - (NB - 1) * NQ
ACC_R = NQ + 8
RZ = 72
ZB = 64

_f32 = jnp.float32


def _nm_body(xr, per, w4r, b4r, w2r, b2r, axr, bxr, dxr, exr, aper, bpr):
    xpe = jnp.concatenate([xr[...], per[...]], axis=1)
    y = jnp.dot(xpe, w4r[...], preferred_element_type=_f32) + b4r[...]
    p = jnp.dot(per[...], w2r[...], preferred_element_type=_f32) + b2r[...]
    axr[...] = y[:, 0:D]
    bxr[...] = y[:, D:2 * D]
    dxr[...] = y[:, 2 * D:3 * D]
    exr[...] = y[:, 3 * D:4 * D]
    aper[...] = p[:, 0:D]
    bpr[...] = p[:, D:2 * D]


def _node_matmul(x, pe, w4, b4, w2, b2):
    b2d = lambda: pl.BlockSpec((BN, D), lambda i: (i, 0))
    outs = [jax.ShapeDtypeStruct((N, D), _f32) for _ in range(6)]
    return pl.pallas_call(
        _nm_body, grid=(N // BN,),
        in_specs=[b2d(), b2d(),
                  pl.BlockSpec((2 * D, 4 * D), lambda i: (0, 0)),
                  pl.BlockSpec((1, 4 * D), lambda i: (0, 0)),
                  pl.BlockSpec((D, 2 * D), lambda i: (0, 0)),
                  pl.BlockSpec((1, 2 * D), lambda i: (0, 0))],
        out_specs=[b2d() for _ in range(6)],
        out_shape=outs,
    )(x, pe, w4, b4, w2, b2)


def _ce0_body(er, wcr, bcr, cer):
    cer[...] = jnp.dot(er[...], wcr[...], preferred_element_type=_f32) + bcr[...]


def _ce0(e, wc, bc):
    return pl.pallas_call(
        _ce0_body, grid=(E // BE,),
        in_specs=[pl.BlockSpec((BE, D), lambda i: (i, 0)),
                  pl.BlockSpec((D, D), lambda i: (0, 0)),
                  pl.BlockSpec((1, D), lambda i: (0, 0))],
        out_specs=[pl.BlockSpec((BE, D), lambda i: (i, 0))],
        out_shape=[jax.ShapeDtypeStruct((E, D), _f32)],
    )(e, wc, bc)[0]


def _ebn_apply(eijr, str_, gr, br, einr):
    mu = str_[0:1, :] * (1.0 / E)
    var = str_[1:2, :] * (1.0 / E) - mu * mu
    xn = (eijr - mu) * lax.rsqrt(var + 1e-5) * gr[...] + br[...]
    return einr[...] + jnp.maximum(xn, 0.0)


def _est_body(eijr, str_):
    i = pl.program_id(0)

    @pl.when(i == 0)
    def _init():
        str_[...] = jnp.zeros_like(str_)

    eij = eijr[...]
    str_[0:1, :] += jnp.sum(eij, axis=0, keepdims=True)
    str_[1:2, :] += jnp.sum(eij * eij, axis=0, keepdims=True)


def _estats(eij):
    return pl.pallas_call(
        _est_body, grid=(E // BE,),
        in_specs=[pl.BlockSpec((BE, D), lambda i: (i, 0))],
        out_specs=[pl.BlockSpec((2, D), lambda i: (0, 0))],
        out_shape=[jax.ShapeDtypeStruct((2, D), _f32)],
    )(eij)[0]


def _efuse_body(eijr, str_, einr, gr, br, wcr, bcr, eor, cer):
    e_out = _ebn_apply(eijr[...], str_, gr, br, einr)
    eor[...] = e_out
    cer[...] = jnp.dot(e_out, wcr[...], preferred_element_type=_f32) + bcr[...]


def _efuse(eij, st, e_in, g, b, wc, bc):
    return pl.pallas_call(
        _efuse_body, grid=(E // BE,),
        in_specs=[pl.BlockSpec((BE, D), lambda i: (i, 0)),
                  pl.BlockSpec((2, D), lambda i: (0, 0)),
                  pl.BlockSpec((BE, D), lambda i: (i, 0)),
                  pl.BlockSpec((1, D), lambda i: (0, 0)),
                  pl.BlockSpec((1, D), lambda i: (0, 0)),
                  pl.BlockSpec((D, D), lambda i: (0, 0)),
                  pl.BlockSpec((1, D), lambda i: (0, 0))],
        out_specs=[pl.BlockSpec((BE, D), lambda i: (i, 0)),
                   pl.BlockSpec((BE, D), lambda i: (i, 0))],
        out_shape=[jax.ShapeDtypeStruct((E, D), _f32),
                   jax.ShapeDtypeStruct((E, D), _f32)],
    )(eij, st, e_in, g, b, wc, bc)


def _efinal_body(eijr, str_, einr, gr, br, eor):
    eor[...] = _ebn_apply(eijr[...], str_, gr, br, einr)


def _efinal(eij, st, e_in, g, b):
    return pl.pallas_call(
        _efinal_body, grid=(E // BE,),
        in_specs=[pl.BlockSpec((BE, D), lambda i: (i, 0)),
                  pl.BlockSpec((2, D), lambda i: (0, 0)),
                  pl.BlockSpec((BE, D), lambda i: (i, 0)),
                  pl.BlockSpec((1, D), lambda i: (0, 0)),
                  pl.BlockSpec((1, D), lambda i: (0, 0))],
        out_specs=[pl.BlockSpec((BE, D), lambda i: (i, 0))],
        out_shape=[jax.ShapeDtypeStruct((E, D), _f32)],
    )(eij, st, e_in, g, b)[0]


def _xst_body(axr, numr, denr, str_):
    i = pl.program_id(0)

    @pl.when(i == 0)
    def _init():
        str_[...] = jnp.zeros_like(str_)

    xpre = axr[...] + numr[...] / (denr[...] + 1e-6)
    str_[0:1, :] += jnp.sum(xpre, axis=0, keepdims=True)
    str_[1:2, :] += jnp.sum(xpre * xpre, axis=0, keepdims=True)


def _xstats(ax, num, den):
    b2d = lambda: pl.BlockSpec((BN, D), lambda i: (i, 0))
    return pl.pallas_call(
        _xst_body, grid=(N // BN,),
        in_specs=[b2d(), b2d(), b2d()],
        out_specs=[pl.BlockSpec((2, D), lambda i: (0, 0))],
        out_shape=[jax.ShapeDtypeStruct((2, D), _f32)],
    )(ax, num, den)[0]


def _xap_body(axr, aper, numr, denr, numpr, str_, xinr, peinr, gr, br, xor_, peor):
    den = denr[...] + 1e-6
    xpre = axr[...] + numr[...] / den
    mu = str_[0:1, :] * (1.0 / N)
    var = str_[1:2, :] * (1.0 / N) - mu * mu
    xn = (xpre - mu) * lax.rsqrt(var + 1e-5) * gr[...] + br[...]
    xor_[...] = xinr[...] + jnp.maximum(xn, 0.0)
    peor[...] = peinr[...] + jnp.tanh(aper[...] + numpr[...] / den)


def _xapply(ax, ape, num, den, nump, xst, x_in, pe_in, g, b):
    b2d = lambda: pl.BlockSpec((BN, D), lambda i: (i, 0))
    return pl.pallas_call(
        _xap_body, grid=(N // BN,),
        in_specs=[b2d(), b2d(), b2d(), b2d(), b2d(),
                  pl.BlockSpec((2, D), lambda i: (0, 0)),
                  b2d(), b2d(),
                  pl.BlockSpec((1, D), lambda i: (0, 0)),
                  pl.BlockSpec((1, D), lambda i: (0, 0))],
        out_specs=[b2d(), b2d()],
        out_shape=[jax.ShapeDtypeStruct((N, D), _f32),
                   jax.ShapeDtypeStruct((N, D), _f32)],
    )(ax, ape, num, den, nump, xst, x_in, pe_in, g, b)


def _sc_edge(pk, offs, dx, ex, bx, bp, ce):
    mesh = plsc.VectorSubcoreMesh(core_axis_name="c", subcore_axis_name="s")
    idx_t = lambda: pltpu.VMEM((C,), jnp.int32)
    row_t = lambda: pltpu.VMEM((C, D), _f32)
    buf_t = [pltpu.VMEM((3 * C,), jnp.int32),
             idx_t(), idx_t(), idx_t(), idx_t(), idx_t(),
             row_t(), row_t(), row_t(), row_t(), row_t(),
             pltpu.SemaphoreType.DMA, pltpu.SemaphoreType.DMA]

    @functools.partial(
        pl.kernel, mesh=mesh,
        out_type=[jax.ShapeDtypeStruct((E + 8, D), _f32),
                  jax.ShapeDtypeStruct((N, D), _f32),
                  jax.ShapeDtypeStruct((N, D), _f32),
                  jax.ShapeDtypeStruct((N, D), _f32)],
        scratch_types=[pltpu.VMEM((16,), jnp.int32),
                       pltpu.VMEM((ZB, D), _f32)]
                      + buf_t + buf_t
                      + [pltpu.VMEM_SHARED((ACC_R, D), _f32),
                         pltpu.VMEM_SHARED((ACC_R, D), _f32),
                         pltpu.VMEM_SHARED((ACC_R, D), _f32)])
    def k(pk_h, offs_h, dx_h, ex_h, bx_h, bp_h, ce_h,
          eij_h, num_h, den_h, nump_h,
          offv, zb,
          pkb0, srcg0, dstl0, dstg0, eids0, eidg0,
          dxb0, exb0, bxb0, bpb0, ceb0, gsm0, ssm0,
          pkb1, srcg1, dstl1, dstg1, eids1, eidg1,
          dxb1, exb1, bxb1, bpb1, ceb1, gsm1, ssm1,
          accn, accd, accp):
        c = lax.axis_index("c")
        s = lax.axis_index("s")
        B = [(pkb0, srcg0, dstl0, dstg0, eids0, eidg0,
              dxb0, exb0, bxb0, bpb0, ceb0, gsm0, ssm0),
             (pkb1, srcg1, dstl1, dstg1, eids1, eidg1,
              dxb1, exb1, bxb1, bpb1, ceb1, gsm1, ssm1)]

        def zrow(r, _):
            for k8 in range(D // 16):
                zb[r, pl.ds(16 * k8, 16)] = jnp.zeros((16,), _f32)
            return 0

        lax.fori_loop(0, ZB, zrow, 0)
        pltpu.sync_copy(offs_h, offv)
        ov = offv[...]

        for p in range(NPH):
            start_b = jnp.where(c == 0, ov[2 * p], ov[2 * (NPH + p)])
            len_b = jnp.where(c == 0, ov[2 * p + 1], ov[2 * (NPH + p) + 1])
            qbase = (NPH * c + p) * NQ
            tlen = len_b // NSUB
            nch = tlen // C
            tstart = start_b + s * tlen

            for (zo, zn) in ((0, ZB), (ZB, RZ - ZB)):
                pltpu.sync_copy(zb.at[pl.ds(0, zn)],
                                accn.at[pl.ds(s * RZ + zo, zn)])
                pltpu.sync_copy(zb.at[pl.ds(0, zn)],
                                accd.at[pl.ds(s * RZ + zo, zn)])
                pltpu.sync_copy(zb.at[pl.ds(0, zn)],
                                accp.at[pl.ds(s * RZ + zo, zn)])

            @pl.when(s == NSUB - 1)
            def _ztail():
                tb = NSUB * RZ
                for (zo, zn) in ((0, ZB), (ZB, ACC_R - NSUB * RZ - ZB)):
                    pltpu.sync_copy(zb.at[pl.ds(0, zn)],
                                    accn.at[pl.ds(tb + zo, zn)])
                    pltpu.sync_copy(zb.at[pl.ds(0, zn)],
                                    accd.at[pl.ds(tb + zo, zn)])
                    pltpu.sync_copy(zb.at[pl.ds(0, zn)],
                                    accp.at[pl.ds(tb + zo, zn)])

            plsc.subcore_barrier()

            def prefetch(base, bi):
                (pkb, srcg, dstl, dstg, eids, eidg,
                 dxb, exb, bxb, bpb, ceb, gsm, _ssm) = B[bi]
                pko = pl.multiple_of(3 * base, 16)
                pltpu.sync_copy(pk_h.at[pl.ds(pko, 3 * C)], pkb)

                def adj(r, _):
                    sl = pl.ds(16 * r, 16)
                    sv = pkb[pl.ds(16 * r, 16)]
                    dv = pkb[pl.ds(C + 16 * r, 16)]
                    ev = pkb[pl.ds(2 * C + 16 * r, 16)]
                    srcg[sl] = sv
                    dstl[sl] = jnp.minimum(dv - qbase, NQ)
                    dstg[sl] = jnp.minimum(dv, N - 1)
                    eids[sl] = ev
                    eidg[sl] = jnp.minimum(ev, E - 1)
                    return 0

                lax.fori_loop(0, C // 16, adj, 0)
                pltpu.async_copy(dx_h.at[dstg], dxb, gsm)
                pltpu.async_copy(ex_h.at[srcg], exb, gsm)
                pltpu.async_copy(bx_h.at[srcg], bxb, gsm)
                pltpu.async_copy(bp_h.at[srcg], bpb, gsm)
                pltpu.async_copy(ce_h.at[eidg], ceb, gsm)

            def wait_gathers(bi):
                (pkb, srcg, dstl, dstg, eids, eidg,
                 dxb, exb, bxb, bpb, ceb, gsm, _ssm) = B[bi]
                pltpu.make_async_copy(dx_h.at[dstg], dxb, gsm).wait()
                pltpu.make_async_copy(ex_h.at[srcg], exb, gsm).wait()
                pltpu.make_async_copy(bx_h.at[srcg], bxb, gsm).wait()
                pltpu.make_async_copy(bp_h.at[srcg], bpb, gsm).wait()
                pltpu.make_async_copy(ce_h.at[eidg], ceb, gsm).wait()

            def compute(bi):
                (pkb, srcg, dstl, dstg, eids, eidg,
                 dxb, exb, bxb, bpb, ceb, _gsm, _ssm) = B[bi]

                def row(r, _):
                    for k8 in range(D // 16):
                        sl = pl.ds(16 * k8, 16)
                        eij = dxb[r, sl] + exb[r, sl] + ceb[r, sl]
                        ceb[r, sl] = eij
                        sg = 1.0 / (1.0 + jnp.exp(-eij))
                        dxb[r, sl] = sg
                        bxb[r, sl] = sg * bxb[r, sl]
                        bpb[r, sl] = sg * bpb[r, sl]
                    return 0

                lax.fori_loop(0, C, row, 0)

            def scatters(bi):
                (pkb, srcg, dstl, dstg, eids, eidg,
                 dxb, exb, bxb, bpb, ceb, _gsm, ssm) = B[bi]
                h = pltpu.async_copy(ceb, eij_h.at[eids], ssm)
                pltpu.sync_copy(bxb, accn.at[dstl], add=True)
                pltpu.sync_copy(dxb, accd.at[dstl], add=True)
                pltpu.sync_copy(bpb, accp.at[dstl], add=True)
                h.wait()

            nch2 = nch // 2

            def steady(jj, _):
                base = pl.multiple_of(tstart + 2 * jj * C, 16)
                prefetch(base + C, 1)
                wait_gathers(0)
                compute(0)
                scatters(0)
                prefetch(base + 2 * C, 0)
                wait_gathers(1)
                compute(1)
                scatters(1)
                return 0

            @pl.when(nch > 0)
            def _run():
                prefetch(pl.multiple_of(tstart, 16), 0)
                lax.fori_loop(0, nch2 - 1, steady, 0)
                tb_ = pl.multiple_of(tstart + (nch - 2) * C, 16)
                prefetch(tb_ + C, 1)
                wait_gathers(0)
                compute(0)
                scatters(0)
                wait_gathers(1)
                compute(1)
                scatters(1)

            plsc.subcore_barrier()

            @pl.when(s < NSUB - 1)
            def _cmain():
                pltpu.sync_copy(accn.at[pl.ds(s * RZ, RZ)],
                                num_h.at[pl.ds(qbase + s * RZ, RZ)])
                pltpu.sync_copy(accd.at[pl.ds(s * RZ, RZ)],
                                den_h.at[pl.ds(qbase + s * RZ, RZ)])
                pltpu.sync_copy(accp.at[pl.ds(s * RZ, RZ)],
                                nump_h.at[pl.ds(qbase + s * RZ, RZ)])

            tb = (NSUB - 1) * RZ
            nfull = NQ - tb
            nlast = N_LAST - tb
            if p < NPH - 1:
                @pl.when(s == NSUB - 1)
                def _ctail():
                    pltpu.sync_copy(accn.at[pl.ds(tb, nfull)],
                                    num_h.at[pl.ds(qbase + tb, nfull)])
                    pltpu.sync_copy(accd.at[pl.ds(tb, nfull)],
                                    den_h.at[pl.ds(qbase + tb, nfull)])
                    pltpu.sync_copy(accp.at[pl.ds(tb, nfull)],
                                    nump_h.at[pl.ds(qbase + tb, nfull)])
            else:
                @pl.when(jnp.logical_and(s == NSUB - 1, c == 0))
                def _ctailf():
                    pltpu.sync_copy(accn.at[pl.ds(tb, nfull)],
                                    num_h.at[pl.ds(qbase + tb, nfull)])
                    pltpu.sync_copy(accd.at[pl.ds(tb, nfull)],
                                    den_h.at[pl.ds(qbase + tb, nfull)])
                    pltpu.sync_copy(accp.at[pl.ds(tb, nfull)],
                                    nump_h.at[pl.ds(qbase + tb, nfull)])

                @pl.when(jnp.logical_and(s == NSUB - 1, c == 1))
                def _ctaill():
                    pltpu.sync_copy(accn.at[pl.ds(tb, nlast)],
                                    num_h.at[pl.ds(qbase + tb, nlast)])
                    pltpu.sync_copy(accd.at[pl.ds(tb, nlast)],
                                    den_h.at[pl.ds(qbase + tb, nlast)])
                    pltpu.sync_copy(accp.at[pl.ds(tb, nlast)],
                                    nump_h.at[pl.ds(qbase + tb, nlast)])

            if p < NPH - 1:
                plsc.subcore_barrier()

    return k(pk, offs, dx, ex, bx, bp, ce)


def _partition_edges(src0, dst0):
    i32 = jnp.int32
    b = jnp.minimum(dst0 // NQ, NB - 1)
    pos = jnp.zeros((E,), i32)
    starts = []
    start = jnp.zeros((), i32)
    for k in range(NB):
        ink = (b == k).astype(i32)
        rank = jnp.cumsum(ink) - ink
        pos = jnp.where(ink == 1, start + rank, pos)
        cnt = jnp.sum(ink)
        plen = ((cnt + PCP - 1) // PCP) * PCP
        starts.append(start)
        starts.append(plen.astype(i32))
        start = start + plen
    src_p = jnp.zeros((E_PAD,), i32).at[pos].set(src0)
    dst_p = jnp.full((E_PAD,), DUMMY_DST, i32).at[pos].set(dst0)
    eid_p = jnp.full((E_PAD,), E, i32).at[pos].set(jnp.arange(E, dtype=i32))
    pk = jnp.stack([src_p.reshape(-1, C), dst_p.reshape(-1, C),
                    eid_p.reshape(-1, C)], axis=1).reshape(-1)
    z = jnp.zeros((), i32)
    offs = jnp.stack(starts + [z] * (16 - len(starts)))
    return pk, offs


def kernel(x, e, edge_index, pe, params):
    src0 = edge_index[0]
    dst0 = edge_index[1]
    pk, offs = _partition_edges(src0, dst0)
    x_cur, pe_cur, e_cur = x, pe, e
    ceT = None
    for l in range(NL):
        w4 = jnp.concatenate([params['WA'][l], params['WB'][l],
                              params['WD'][l], params['WE'][l]], axis=1)
        b4 = jnp.concatenate([params['bA'][l], params['bB'][l],
                              params['bD'][l], params['bE'][l]], axis=0)[None, :]
        w2 = jnp.concatenate([params['WAp'][l], params['WBp'][l]], axis=1)
        b2 = jnp.concatenate([params['bAp'][l], params['bBp'][l]], axis=0)[None, :]
        if l == 0:
            ceT = _ce0(e_cur, params['WC'][0], params['bC'][0][None, :])
        ax, bx, dx, ex, ape, bp = _node_matmul(x_cur, pe_cur, w4, b4, w2, b2)
        eijF, num, den, nump = _sc_edge(pk, offs, dx, ex, bx, bp, ceT)
        xst = _xstats(ax, num, den)
        x_new, pe_new = _xapply(ax, ape, num, den, nump, xst, x_cur, pe_cur,
                                params['bn_x_g'][l][None, :], params['bn_x_b'][l][None, :])
        st = _estats(eijF)
        g = params['bn_e_g'][l][None, :]
        b = params['bn_e_b'][l][None, :]
        if l < NL - 1:
            e_new, ceT = _efuse(eijF, st, e_cur,
                                g, b, params['WC'][l + 1], params['bC'][l + 1][None, :])
        else:
            e_new = _efinal(eijF, st, e_cur, g, b)
        x_cur, pe_cur, e_cur = x_new, pe_new, e_new
    return (x_cur, e_cur)
